# baseline (speedup 1.0000x reference)
import jax
import jax.numpy as jnp
from jax import lax
from jax.experimental import pallas as pl
from jax.experimental.pallas import tpu as pltpu
from jax.experimental.pallas import tpu_sc as plsc

N_NODES = 100000
N_REL = 32
DIM = 128
E = 500000
MARGIN = 1.0

C = 32
NC, NS, L = 2, 16, 16
NW = NC * NS
NCHUNK = E // C
M = (NCHUNK + NW - 1) // NW
NCHUNK_PAD = M * NW
LC = DIM // L


def _body(comb_h, ridx_h, entc_h, relc_h, out_h,
          idx_v, ridx_v, rows_v, relc_v, stage_v,
          sg0, sg1, si0, si1):
    wid = lax.axis_index("s") * NC + lax.axis_index("c")
    sgs = (sg0, sg1)
    sis = (si0, si1)

    pltpu.sync_copy(relc_h, relc_v)

    def chunk_of(m):
        return wid + NW * m

    def issue_idx(m, B):
        c = chunk_of(m)
        pltpu.async_copy(comb_h.at[c], idx_v.at[B], sis[B])
        pltpu.async_copy(ridx_h.at[c], ridx_v.at[B], sis[B])

    def drain_idx(B):
        pltpu.make_async_copy(comb_h.at[0], idx_v.at[B], sis[B]).wait()
        pltpu.make_async_copy(ridx_h.at[0], ridx_v.at[B], sis[B]).wait()

    def issue_gath(B):
        for a in range(4):
            pltpu.async_copy(entc_h.at[idx_v.at[B, a]], rows_v.at[B, a],
                             sgs[B])

    def drain_gath(B):
        for a in range(4):
            pltpu.make_async_copy(entc_h.at[idx_v.at[B, a]], rows_v.at[B, a],
                                  sgs[B]).wait()

    def compute(m, B, carry):
        def edge(j, car):
            mg, npj, nen, nrl = car
            rv = ridx_v[B, j, :]
            rp_i = rv[0]
            rn_i = rv[1]

            def unp(x32):
                return plsc.unpack(plsc.bitcast(x32, jnp.bfloat16),
                                   format=plsc.PackFormat.INTERLEAVED,
                                   preferred_element_type=jnp.float32)

            def triple(hs, ts, r):
                z = jnp.zeros((L,), jnp.float32)
                sh = st = ehrp = etrp = neh = net = nre = rpn = z
                bases = []
                rps = []
                for q in range(LC // 2):
                    sle = pl.ds(q * L, L)
                    slp = pl.ds(DIM // 2 + q * L, L)
                    ehu = unp(rows_v[B, hs, j, sle])
                    phu = unp(rows_v[B, hs, j, slp])
                    etu = unp(rows_v[B, ts, j, sle])
                    ptu = unp(rows_v[B, ts, j, slp])
                    reu = unp(relc_v[r, sle])
                    rpu = unp(relc_v[r, slp])
                    for h in range(2):
                        eh = ehu[h]
                        ph_ = phu[h]
                        et = etu[h]
                        pt_ = ptu[h]
                        re_ = reu[h]
                        rp_ = rpu[h]
                        sh = sh + eh * ph_
                        st = st + et * pt_
                        ehrp = ehrp + eh * rp_
                        etrp = etrp + et * rp_
                        neh = neh + eh * eh
                        net = net + et * et
                        nre = nre + re_ * re_
                        rpn = rpn + rp_ * rp_
                        bases.append(eh - et + re_)
                        rps.append(rp_)
                shs = jnp.sum(sh)
                sts = jnp.sum(st)
                ehs = jnp.sum(ehrp)
                ets = jnp.sum(etrp)
                nehs = jnp.sum(neh)
                nets = jnp.sum(net)
                nres = jnp.sum(nre)
                rpns = jnp.sum(rpn)
                k = shs - sts
                kv = jnp.zeros((L,), jnp.float32) + k
                sv = jnp.zeros((L,), jnp.float32)
                for q in range(LC):
                    sv = sv + jnp.abs(bases[q] + kv * rps[q])
                score = jnp.sum(sv)
                nh2 = nehs + 2.0 * shs * ehs + shs * shs * rpns
                nt2 = nets + 2.0 * sts * ets + sts * sts * rpns
                return score, nh2, nt2, nehs, nets, nres

            ps_, pnh, pnt, pneh, pnet, pnre = triple(0, 1, rp_i)
            ns_, nnh, nnt, nneh, nnet, nnre = triple(2, 3, rn_i)
            mg = mg + jnp.maximum(ps_ - ns_ + MARGIN, 0.0)
            npj = npj + (jnp.maximum(pnh - 1.0, 0.0)
                         + jnp.maximum(pnt - 1.0, 0.0)
                         + jnp.maximum(nnh - 1.0, 0.0)
                         + jnp.maximum(nnt - 1.0, 0.0))
            nen = nen + (jnp.maximum(pneh - 1.0, 0.0)
                         + jnp.maximum(pnet - 1.0, 0.0)
                         + jnp.maximum(nneh - 1.0, 0.0)
                         + jnp.maximum(nnet - 1.0, 0.0))
            nrl = nrl + (jnp.maximum(pnre - 1.0, 0.0)
                         + jnp.maximum(nnre - 1.0, 0.0))
            return (mg, npj, nen, nrl)

        z = jnp.float32(0.0)
        cm, cp, ce, cr = lax.fori_loop(0, C, edge, (z, z, z, z), unroll=4)
        valid = chunk_of(m) < NCHUNK
        mg, npj, nen, nrl = carry
        return (mg + jnp.where(valid, cm, 0.0),
                npj + jnp.where(valid, cp, 0.0),
                nen + jnp.where(valid, ce, 0.0),
                nrl + jnp.where(valid, cr, 0.0))

    issue_idx(0, 0)
    drain_idx(0)
    issue_gath(0)
    issue_idx(1, 1)

    z = jnp.float32(0.0)

    def pair(mp, carry):
        m0 = 2 * mp
        m1 = m0 + 1
        drain_idx(1)
        issue_gath(1)
        drain_gath(0)
        issue_idx(m0 + 2, 0)
        carry = compute(m0, 0, carry)
        drain_idx(0)
        issue_gath(0)
        drain_gath(1)

        @pl.when(m1 + 2 < M)
        def _():
            issue_idx(m1 + 2, 1)

        carry = compute(m1, 1, carry)
        return carry

    carry = lax.fori_loop(0, M // 2, pair, (z, z, z, z))
    drain_gath(0)
    carry = compute(M - 1, 0, carry)

    iot = jnp.arange(L, dtype=jnp.int32)
    vec = jnp.where(iot == 0, carry[0], jnp.zeros((L,), jnp.float32))
    vec = jnp.where(iot == 1, carry[1], vec)
    vec = jnp.where(iot == 2, carry[2], vec)
    vec = jnp.where(iot == 3, carry[3], vec)
    stage_v[...] = vec
    pltpu.sync_copy(stage_v, out_h.at[wid])


def kernel(entities_user_emb, rel_emb, entities_user_proj_emb, rel_proj_emb,
           pos_h, pos_t, pos_r, neg_h, neg_t, neg_r):
    pad = NCHUNK_PAD * C - E

    def padi(a):
        return jnp.concatenate([a, jnp.zeros((pad,), jnp.int32)])

    comb = jnp.stack([padi(pos_h), padi(pos_t),
                      padi(neg_h), padi(neg_t)])
    comb = comb.reshape(4, NCHUNK_PAD, C).transpose(1, 0, 2)
    ridx = jnp.stack([padi(pos_r), padi(neg_r)], axis=-1)
    ridx = jnp.pad(ridx, ((0, 0), (0, L - 2)))
    ridx = ridx.reshape(NCHUNK_PAD, C, L)

    def pack32(a, b):
        ab = jax.lax.bitcast_convert_type(
            a.astype(jnp.bfloat16).reshape(-1, DIM // 2, 2), jnp.int32)
        bb = jax.lax.bitcast_convert_type(
            b.astype(jnp.bfloat16).reshape(-1, DIM // 2, 2), jnp.int32)
        return jnp.concatenate([ab, bb], axis=1)

    entc = pack32(entities_user_emb, entities_user_proj_emb)
    relc = pack32(rel_emb, rel_proj_emb)

    mesh = plsc.VectorSubcoreMesh(core_axis_name="c", subcore_axis_name="s",
                                  num_cores=NC, num_subcores=NS)
    parts = pl.kernel(
        _body,
        out_type=jax.ShapeDtypeStruct((NW, L), jnp.float32),
        mesh=mesh,
        compiler_params=pltpu.CompilerParams(needs_layout_passes=False),
        scratch_types=[
            pltpu.VMEM((2, 4, C), jnp.int32),
            pltpu.VMEM((2, C, L), jnp.int32),
            pltpu.VMEM((2, 4, C, DIM), jnp.int32),
            pltpu.VMEM((N_REL, DIM), jnp.int32),
            pltpu.VMEM((L,), jnp.float32),
            pltpu.SemaphoreType.DMA,
            pltpu.SemaphoreType.DMA,
            pltpu.SemaphoreType.DMA,
            pltpu.SemaphoreType.DMA,
        ],
    )(comb, ridx, entc, relc)

    s = jnp.sum(parts, axis=0)
    Ef = jnp.float32(E)
    return (s[0] + s[1]) / Ef + s[2] / (4.0 * Ef) + s[3] / (2.0 * Ef)


# speedup vs baseline: 1.4331x; 1.4331x over previous
import jax
import jax.numpy as jnp
from jax import lax
from jax.experimental import pallas as pl
from jax.experimental.pallas import tpu as pltpu
from jax.experimental.pallas import tpu_sc as plsc

N_NODES = 100000
N_REL = 32
DIM = 128
E = 500000
MARGIN = 1.0

C = 32
NC, NS, L = 2, 16, 16
NW = NC * NS
NCHUNK = E // C
M = (NCHUNK + NW - 1) // NW
NCHUNK_PAD = M * NW
LC = DIM // L


def _body(comb_h, ridx_h, ent_h, proj_h, rel_h, relp_h, out_h,
          idx_v, ridx_v, rows_v, rel_v, relp_v, stage_v,
          sg0, sg1, si0, si1):
    wid = lax.axis_index("s") * NC + lax.axis_index("c")
    sgs = (sg0, sg1)
    sis = (si0, si1)

    pltpu.sync_copy(rel_h, rel_v)
    pltpu.sync_copy(relp_h, relp_v)

    def chunk_of(m):
        return wid + NW * m

    def issue_idx(m, B):
        c = chunk_of(m)
        pltpu.async_copy(comb_h.at[c], idx_v.at[B], sis[B])
        pltpu.async_copy(ridx_h.at[c], ridx_v.at[B], sis[B])

    def drain_idx(B):
        pltpu.make_async_copy(comb_h.at[0], idx_v.at[B], sis[B]).wait()
        pltpu.make_async_copy(ridx_h.at[0], ridx_v.at[B], sis[B]).wait()

    GATH = ((ent_h, 0, 0), (proj_h, 0, 1), (ent_h, 1, 2), (proj_h, 1, 3),
            (ent_h, 2, 4), (proj_h, 2, 5), (ent_h, 3, 6), (proj_h, 3, 7))

    def issue_gath(B):
        for tbl, a, t in GATH:
            pltpu.async_copy(tbl.at[idx_v.at[B, a]], rows_v.at[B, t], sgs[B])

    def drain_gath(B):
        for tbl, a, t in GATH:
            pltpu.make_async_copy(tbl.at[idx_v.at[B, a]], rows_v.at[B, t],
                                  sgs[B]).wait()

    def compute(m, B, carry):
        def edge(j, car):
            mg, npj, nen, nrl = car
            rv = ridx_v[B, j, :]
            rp_i = rv[0]
            rn_i = rv[1]

            def triple(s, r):
                z = jnp.zeros((L,), jnp.float32)
                sh = st = ehrp = etrp = neh = net = nre = rpn = z
                bases = []
                rps = []
                for q in range(LC):
                    sl = pl.ds(q * L, L)
                    eh = rows_v[B, s + 0, j, sl]
                    ph_ = rows_v[B, s + 1, j, sl]
                    et = rows_v[B, s + 2, j, sl]
                    pt_ = rows_v[B, s + 3, j, sl]
                    re_ = rel_v[r, sl]
                    rp_ = relp_v[r, sl]
                    sh = sh + eh * ph_
                    st = st + et * pt_
                    ehrp = ehrp + eh * rp_
                    etrp = etrp + et * rp_
                    neh = neh + eh * eh
                    net = net + et * et
                    nre = nre + re_ * re_
                    rpn = rpn + rp_ * rp_
                    bases.append(eh - et + re_)
                    rps.append(rp_)
                shs = jnp.sum(sh)
                sts = jnp.sum(st)
                ehs = jnp.sum(ehrp)
                ets = jnp.sum(etrp)
                nehs = jnp.sum(neh)
                nets = jnp.sum(net)
                nres = jnp.sum(nre)
                rpns = jnp.sum(rpn)
                k = shs - sts
                kv = jnp.zeros((L,), jnp.float32) + k
                sv = jnp.zeros((L,), jnp.float32)
                for q in range(LC):
                    sv = sv + jnp.abs(bases[q] + kv * rps[q])
                score = jnp.sum(sv)
                nh2 = nehs + 2.0 * shs * ehs + shs * shs * rpns
                nt2 = nets + 2.0 * sts * ets + sts * sts * rpns
                return score, nh2, nt2, nehs, nets, nres

            ps_, pnh, pnt, pneh, pnet, pnre = triple(0, rp_i)
            ns_, nnh, nnt, nneh, nnet, nnre = triple(4, rn_i)
            mg = mg + jnp.maximum(ps_ - ns_ + MARGIN, 0.0)
            npj = npj + (jnp.maximum(pnh - 1.0, 0.0)
                         + jnp.maximum(pnt - 1.0, 0.0)
                         + jnp.maximum(nnh - 1.0, 0.0)
                         + jnp.maximum(nnt - 1.0, 0.0))
            nen = nen + (jnp.maximum(pneh - 1.0, 0.0)
                         + jnp.maximum(pnet - 1.0, 0.0)
                         + jnp.maximum(nneh - 1.0, 0.0)
                         + jnp.maximum(nnet - 1.0, 0.0))
            nrl = nrl + (jnp.maximum(pnre - 1.0, 0.0)
                         + jnp.maximum(nnre - 1.0, 0.0))
            return (mg, npj, nen, nrl)

        z = jnp.float32(0.0)
        cm, cp, ce, cr = plsc.parallel_loop(
            0, C, 1, unroll=4, carry=(z, z, z, z))(edge)
        valid = chunk_of(m) < NCHUNK
        mg, npj, nen, nrl = carry
        return (mg + jnp.where(valid, cm, 0.0),
                npj + jnp.where(valid, cp, 0.0),
                nen + jnp.where(valid, ce, 0.0),
                nrl + jnp.where(valid, cr, 0.0))

    issue_idx(0, 0)
    drain_idx(0)
    issue_gath(0)
    issue_idx(1, 1)

    z = jnp.float32(0.0)

    def pair(mp, carry):
        m0 = 2 * mp
        m1 = m0 + 1
        drain_idx(1)
        issue_gath(1)
        drain_gath(0)
        issue_idx(m0 + 2, 0)
        carry = compute(m0, 0, carry)
        drain_idx(0)
        issue_gath(0)
        drain_gath(1)

        @pl.when(m1 + 2 < M)
        def _():
            issue_idx(m1 + 2, 1)

        carry = compute(m1, 1, carry)
        return carry

    carry = lax.fori_loop(0, M // 2, pair, (z, z, z, z))
    drain_gath(0)
    carry = compute(M - 1, 0, carry)

    iot = jnp.arange(L, dtype=jnp.int32)
    vec = jnp.where(iot == 0, carry[0], jnp.zeros((L,), jnp.float32))
    vec = jnp.where(iot == 1, carry[1], vec)
    vec = jnp.where(iot == 2, carry[2], vec)
    vec = jnp.where(iot == 3, carry[3], vec)
    stage_v[...] = vec
    pltpu.sync_copy(stage_v, out_h.at[wid])


def kernel(entities_user_emb, rel_emb, entities_user_proj_emb, rel_proj_emb,
           pos_h, pos_t, pos_r, neg_h, neg_t, neg_r):
    pad = NCHUNK_PAD * C - E

    def padi(a):
        return jnp.concatenate([a, jnp.zeros((pad,), jnp.int32)])

    comb = jnp.stack([padi(pos_h), padi(pos_t),
                      padi(neg_h), padi(neg_t)])
    comb = comb.reshape(4, NCHUNK_PAD, C).transpose(1, 0, 2)
    ridx = jnp.stack([padi(pos_r), padi(neg_r)], axis=-1)
    ridx = jnp.pad(ridx, ((0, 0), (0, L - 2)))
    ridx = ridx.reshape(NCHUNK_PAD, C, L)

    mesh = plsc.VectorSubcoreMesh(core_axis_name="c", subcore_axis_name="s",
                                  num_cores=NC, num_subcores=NS)
    parts = pl.kernel(
        _body,
        out_type=jax.ShapeDtypeStruct((NW, L), jnp.float32),
        mesh=mesh,
        compiler_params=pltpu.CompilerParams(needs_layout_passes=False),
        scratch_types=[
            pltpu.VMEM((2, 4, C), jnp.int32),
            pltpu.VMEM((2, C, L), jnp.int32),
            pltpu.VMEM((2, 8, C, DIM), jnp.float32),
            pltpu.VMEM((N_REL, DIM), jnp.float32),
            pltpu.VMEM((N_REL, DIM), jnp.float32),
            pltpu.VMEM((L,), jnp.float32),
            pltpu.SemaphoreType.DMA,
            pltpu.SemaphoreType.DMA,
            pltpu.SemaphoreType.DMA,
            pltpu.SemaphoreType.DMA,
        ],
    )(comb, ridx, entities_user_emb, entities_user_proj_emb,
      rel_emb, rel_proj_emb)

    s = jnp.sum(parts, axis=0)
    Ef = jnp.float32(E)
    return (s[0] + s[1]) / Ef + s[2] / (4.0 * Ef) + s[3] / (2.0 * Ef)


# speedup vs baseline: 1.5772x; 1.1005x over previous
import jax
import jax.numpy as jnp
from jax import lax
from jax.experimental import pallas as pl
from jax.experimental.pallas import tpu as pltpu
from jax.experimental.pallas import tpu_sc as plsc

N_NODES = 100000
N_REL = 32
DIM = 128
E = 500000
MARGIN = 1.0

C = 32
NC, NS, L = 2, 16, 16
NW = NC * NS
NCHUNK = E // C
M = (NCHUNK + NW - 1) // NW
NCHUNK_PAD = M * NW
LC = DIM // L


def _body(comb_h, ridx_h, ent_h, proj_h, rel_h, relp_h, out_h,
          idx_v, ridx_v, rows_v, rel_v, relp_v, meta_v, stage_v,
          sg0, sg1, si0, si1):
    wid = lax.axis_index("s") * NC + lax.axis_index("c")
    sgs = (sg0, sg1)
    sis = (si0, si1)

    pltpu.sync_copy(rel_h, rel_v)
    pltpu.sync_copy(relp_h, relp_v)

    iot = jnp.arange(L, dtype=jnp.int32)
    zl = jnp.zeros((L,), jnp.float32)
    for r in range(N_REL):
        accp = zl
        acce = zl
        for q in range(LC):
            sl = pl.ds(q * L, L)
            a = relp_v[r, sl]
            b = rel_v[r, sl]
            accp = accp + a * a
            acce = acce + b * b
        vecm = jnp.where(iot == 0, jnp.sum(accp), zl)
        vecm = jnp.where(iot == 1,
                         jnp.maximum(jnp.sum(acce) - 1.0, 0.0), vecm)
        meta_v[r] = vecm

    def chunk_of(m):
        return wid + NW * m

    def issue_idx(m, B):
        c = chunk_of(m)
        pltpu.async_copy(comb_h.at[c], idx_v.at[B], sis[B])
        pltpu.async_copy(ridx_h.at[c], ridx_v.at[B], sis[B])

    def drain_idx(B):
        pltpu.make_async_copy(comb_h.at[0], idx_v.at[B], sis[B]).wait()
        pltpu.make_async_copy(ridx_h.at[0], ridx_v.at[B], sis[B]).wait()

    GATH = ((ent_h, 0, 0), (proj_h, 0, 1), (ent_h, 1, 2), (proj_h, 1, 3),
            (ent_h, 2, 4), (proj_h, 2, 5), (ent_h, 3, 6), (proj_h, 3, 7))

    def issue_gath(B):
        for tbl, a, t in GATH:
            pltpu.async_copy(tbl.at[idx_v.at[B, a]], rows_v.at[B, t], sgs[B])

    def drain_gath(B):
        for tbl, a, t in GATH:
            pltpu.make_async_copy(tbl.at[idx_v.at[B, a]], rows_v.at[B, t],
                                  sgs[B]).wait()

    def compute(m, B, carry):
        def edge(j, car):
            mg, npj, nen, nrl = car
            rv = ridx_v[B, j, :]
            rp_i = rv[0]
            rn_i = rv[1]

            def triple(s, r):
                z = jnp.zeros((L,), jnp.float32)
                sh = st = ehrp = etrp = neh = net = z
                bases = []
                rps = []
                for q in range(LC):
                    sl = pl.ds(q * L, L)
                    eh = rows_v[B, s + 0, j, sl]
                    ph_ = rows_v[B, s + 1, j, sl]
                    et = rows_v[B, s + 2, j, sl]
                    pt_ = rows_v[B, s + 3, j, sl]
                    re_ = rel_v[r, sl]
                    rp_ = relp_v[r, sl]
                    sh = sh + eh * ph_
                    st = st + et * pt_
                    ehrp = ehrp + eh * rp_
                    etrp = etrp + et * rp_
                    neh = neh + eh * eh
                    net = net + et * et
                    bases.append(eh - et + re_)
                    rps.append(rp_)
                shs = jnp.sum(sh)
                sts = jnp.sum(st)
                ehs = jnp.sum(ehrp)
                ets = jnp.sum(etrp)
                nehs = jnp.sum(neh)
                nets = jnp.sum(net)
                mrow = meta_v[r, :]
                rpns = mrow[0]
                nres = mrow[1]
                k = shs - sts
                kv = jnp.zeros((L,), jnp.float32) + k
                sv = jnp.zeros((L,), jnp.float32)
                for q in range(LC):
                    sv = sv + jnp.abs(bases[q] + kv * rps[q])
                score = jnp.sum(sv)
                nh2 = nehs + 2.0 * shs * ehs + shs * shs * rpns
                nt2 = nets + 2.0 * sts * ets + sts * sts * rpns
                return score, nh2, nt2, nehs, nets, nres

            ps_, pnh, pnt, pneh, pnet, pnre = triple(0, rp_i)
            ns_, nnh, nnt, nneh, nnet, nnre = triple(4, rn_i)
            mg = mg + jnp.maximum(ps_ - ns_ + MARGIN, 0.0)
            npj = npj + (jnp.maximum(pnh - 1.0, 0.0)
                         + jnp.maximum(pnt - 1.0, 0.0)
                         + jnp.maximum(nnh - 1.0, 0.0)
                         + jnp.maximum(nnt - 1.0, 0.0))
            nen = nen + (jnp.maximum(pneh - 1.0, 0.0)
                         + jnp.maximum(pnet - 1.0, 0.0)
                         + jnp.maximum(nneh - 1.0, 0.0)
                         + jnp.maximum(nnet - 1.0, 0.0))
            nrl = nrl + pnre + nnre
            return (mg, npj, nen, nrl)

        z = jnp.float32(0.0)
        cm, cp, ce, cr = plsc.parallel_loop(
            0, C, 1, unroll=4, carry=(z, z, z, z))(edge)
        valid = chunk_of(m) < NCHUNK
        mg, npj, nen, nrl = carry
        return (mg + jnp.where(valid, cm, 0.0),
                npj + jnp.where(valid, cp, 0.0),
                nen + jnp.where(valid, ce, 0.0),
                nrl + jnp.where(valid, cr, 0.0))

    issue_idx(0, 0)
    drain_idx(0)
    issue_gath(0)
    issue_idx(1, 1)

    z = jnp.float32(0.0)

    def pair(mp, carry):
        m0 = 2 * mp
        m1 = m0 + 1
        drain_idx(1)
        issue_gath(1)
        drain_gath(0)
        issue_idx(m0 + 2, 0)
        carry = compute(m0, 0, carry)
        drain_idx(0)
        issue_gath(0)
        drain_gath(1)

        @pl.when(m1 + 2 < M)
        def _():
            issue_idx(m1 + 2, 1)

        carry = compute(m1, 1, carry)
        return carry

    carry = lax.fori_loop(0, M // 2, pair, (z, z, z, z))
    drain_gath(0)
    carry = compute(M - 1, 0, carry)

    vec = jnp.where(iot == 0, carry[0], jnp.zeros((L,), jnp.float32))
    vec = jnp.where(iot == 1, carry[1], vec)
    vec = jnp.where(iot == 2, carry[2], vec)
    vec = jnp.where(iot == 3, carry[3], vec)
    stage_v[...] = vec
    pltpu.sync_copy(stage_v, out_h.at[wid])


def kernel(entities_user_emb, rel_emb, entities_user_proj_emb, rel_proj_emb,
           pos_h, pos_t, pos_r, neg_h, neg_t, neg_r):
    pad = NCHUNK_PAD * C - E

    def padi(a):
        return jnp.concatenate([a, jnp.zeros((pad,), jnp.int32)])

    comb = jnp.stack([padi(pos_h), padi(pos_t),
                      padi(neg_h), padi(neg_t)])
    comb = comb.reshape(4, NCHUNK_PAD, C).transpose(1, 0, 2)
    ridx = jnp.stack([padi(pos_r), padi(neg_r)], axis=-1)
    ridx = jnp.pad(ridx, ((0, 0), (0, L - 2)))
    ridx = ridx.reshape(NCHUNK_PAD, C, L)

    mesh = plsc.VectorSubcoreMesh(core_axis_name="c", subcore_axis_name="s",
                                  num_cores=NC, num_subcores=NS)
    parts = pl.kernel(
        _body,
        out_type=jax.ShapeDtypeStruct((NW, L), jnp.float32),
        mesh=mesh,
        compiler_params=pltpu.CompilerParams(needs_layout_passes=False),
        scratch_types=[
            pltpu.VMEM((2, 4, C), jnp.int32),
            pltpu.VMEM((2, C, L), jnp.int32),
            pltpu.VMEM((2, 8, C, DIM), jnp.float32),
            pltpu.VMEM((N_REL, DIM), jnp.float32),
            pltpu.VMEM((N_REL, DIM), jnp.float32),
            pltpu.VMEM((N_REL, L), jnp.float32),
            pltpu.VMEM((L,), jnp.float32),
            pltpu.SemaphoreType.DMA,
            pltpu.SemaphoreType.DMA,
            pltpu.SemaphoreType.DMA,
            pltpu.SemaphoreType.DMA,
        ],
    )(comb, ridx, entities_user_emb, entities_user_proj_emb,
      rel_emb, rel_proj_emb)

    s = jnp.sum(parts, axis=0)
    Ef = jnp.float32(E)
    return (s[0] + s[1]) / Ef + s[2] / (4.0 * Ef) + s[3] / (2.0 * Ef)


# speedup vs baseline: 1.6034x; 1.0166x over previous
import jax
import jax.numpy as jnp
from jax import lax
from jax.experimental import pallas as pl
from jax.experimental.pallas import tpu as pltpu
from jax.experimental.pallas import tpu_sc as plsc

N_NODES = 100000
N_REL = 32
DIM = 128
E = 500000
MARGIN = 1.0

C = 40
NC, NS, L = 2, 16, 16
NW = NC * NS
NCHUNK = E // C
M = (NCHUNK + NW - 1) // NW
NCHUNK_PAD = M * NW
LC = DIM // L


def _body(comb_h, ridx_h, ent_h, proj_h, rel_h, relp_h, out_h,
          idx_v, ridx_v, rows_v, rel_v, relp_v, meta_v, stage_v,
          sg0, sg1, si0, si1):
    wid = lax.axis_index("s") * NC + lax.axis_index("c")
    sgs = (sg0, sg1)
    sis = (si0, si1)

    pltpu.sync_copy(rel_h, rel_v)
    pltpu.sync_copy(relp_h, relp_v)

    iot = jnp.arange(L, dtype=jnp.int32)
    zl = jnp.zeros((L,), jnp.float32)
    for r in range(N_REL):
        accp = zl
        acce = zl
        for q in range(LC):
            sl = pl.ds(q * L, L)
            a = relp_v[r, sl]
            b = rel_v[r, sl]
            accp = accp + a * a
            acce = acce + b * b
        vecm = jnp.where(iot == 0, jnp.sum(accp), zl)
        vecm = jnp.where(iot == 1,
                         jnp.maximum(jnp.sum(acce) - 1.0, 0.0), vecm)
        meta_v[r] = vecm

    def chunk_of(m):
        return wid + NW * m

    def issue_idx(m, B):
        c = chunk_of(m)
        pltpu.async_copy(comb_h.at[c], idx_v.at[B], sis[B])
        pltpu.async_copy(ridx_h.at[c], ridx_v.at[B], sis[B])

    def drain_idx(B):
        pltpu.make_async_copy(comb_h.at[0], idx_v.at[B], sis[B]).wait()
        pltpu.make_async_copy(ridx_h.at[0], ridx_v.at[B], sis[B]).wait()

    GATH = ((ent_h, 0, 0), (proj_h, 0, 1), (ent_h, 1, 2), (proj_h, 1, 3),
            (ent_h, 2, 4), (proj_h, 2, 5), (ent_h, 3, 6), (proj_h, 3, 7))

    def issue_gath(B):
        for tbl, a, t in GATH:
            pltpu.async_copy(tbl.at[idx_v.at[B, a]], rows_v.at[B, t], sgs[B])

    def drain_gath(B):
        for tbl, a, t in GATH:
            pltpu.make_async_copy(tbl.at[idx_v.at[B, a]], rows_v.at[B, t],
                                  sgs[B]).wait()

    def compute(m, B, carry):
        def edge(j, car):
            mg, npj, nen, nrl = car
            rv = ridx_v[B, j, :]
            rp_i = rv[0]
            rn_i = rv[1]

            def triple(s, r):
                z = jnp.zeros((L,), jnp.float32)
                sh = st = ehrp = etrp = neh = net = z
                bases = []
                rps = []
                for q in range(LC):
                    sl = pl.ds(q * L, L)
                    eh = rows_v[B, s + 0, j, sl]
                    ph_ = rows_v[B, s + 1, j, sl]
                    et = rows_v[B, s + 2, j, sl]
                    pt_ = rows_v[B, s + 3, j, sl]
                    re_ = rel_v[r, sl]
                    rp_ = relp_v[r, sl]
                    sh = sh + eh * ph_
                    st = st + et * pt_
                    ehrp = ehrp + eh * rp_
                    etrp = etrp + et * rp_
                    neh = neh + eh * eh
                    net = net + et * et
                    bases.append(eh - et + re_)
                    rps.append(rp_)
                shs = jnp.sum(sh)
                sts = jnp.sum(st)
                ehs = jnp.sum(ehrp)
                ets = jnp.sum(etrp)
                nehs = jnp.sum(neh)
                nets = jnp.sum(net)
                mrow = meta_v[r, :]
                rpns = mrow[0]
                nres = mrow[1]
                k = shs - sts
                kv = jnp.zeros((L,), jnp.float32) + k
                sv = jnp.zeros((L,), jnp.float32)
                for q in range(LC):
                    sv = sv + jnp.abs(bases[q] + kv * rps[q])
                score = jnp.sum(sv)
                nh2 = nehs + 2.0 * shs * ehs + shs * shs * rpns
                nt2 = nets + 2.0 * sts * ets + sts * sts * rpns
                return score, nh2, nt2, nehs, nets, nres

            ps_, pnh, pnt, pneh, pnet, pnre = triple(0, rp_i)
            ns_, nnh, nnt, nneh, nnet, nnre = triple(4, rn_i)
            mg = mg + jnp.maximum(ps_ - ns_ + MARGIN, 0.0)
            npj = npj + (jnp.maximum(pnh - 1.0, 0.0)
                         + jnp.maximum(pnt - 1.0, 0.0)
                         + jnp.maximum(nnh - 1.0, 0.0)
                         + jnp.maximum(nnt - 1.0, 0.0))
            nen = nen + (jnp.maximum(pneh - 1.0, 0.0)
                         + jnp.maximum(pnet - 1.0, 0.0)
                         + jnp.maximum(nneh - 1.0, 0.0)
                         + jnp.maximum(nnet - 1.0, 0.0))
            nrl = nrl + pnre + nnre
            return (mg, npj, nen, nrl)

        z = jnp.float32(0.0)
        cm, cp, ce, cr = plsc.parallel_loop(
            0, C, 1, unroll=4, carry=(z, z, z, z))(edge)
        valid = chunk_of(m) < NCHUNK
        mg, npj, nen, nrl = carry
        return (mg + jnp.where(valid, cm, 0.0),
                npj + jnp.where(valid, cp, 0.0),
                nen + jnp.where(valid, ce, 0.0),
                nrl + jnp.where(valid, cr, 0.0))

    issue_idx(0, 0)
    drain_idx(0)
    issue_gath(0)
    issue_idx(1, 1)

    z = jnp.float32(0.0)

    def pair(mp, carry):
        m0 = 2 * mp
        m1 = m0 + 1
        drain_idx(1)
        issue_gath(1)
        drain_gath(0)
        issue_idx(m0 + 2, 0)
        carry = compute(m0, 0, carry)
        drain_idx(0)
        issue_gath(0)
        drain_gath(1)

        @pl.when(m1 + 2 < M)
        def _():
            issue_idx(m1 + 2, 1)

        carry = compute(m1, 1, carry)
        return carry

    carry = lax.fori_loop(0, M // 2, pair, (z, z, z, z))
    drain_gath(0)
    carry = compute(M - 1, 0, carry)

    vec = jnp.where(iot == 0, carry[0], jnp.zeros((L,), jnp.float32))
    vec = jnp.where(iot == 1, carry[1], vec)
    vec = jnp.where(iot == 2, carry[2], vec)
    vec = jnp.where(iot == 3, carry[3], vec)
    stage_v[...] = vec
    pltpu.sync_copy(stage_v, out_h.at[wid])


def kernel(entities_user_emb, rel_emb, entities_user_proj_emb, rel_proj_emb,
           pos_h, pos_t, pos_r, neg_h, neg_t, neg_r):
    pad = NCHUNK_PAD * C - E

    def padi(a):
        return jnp.concatenate([a, jnp.zeros((pad,), jnp.int32)])

    comb = jnp.stack([padi(pos_h), padi(pos_t),
                      padi(neg_h), padi(neg_t)])
    comb = comb.reshape(4, NCHUNK_PAD, C).transpose(1, 0, 2)
    ridx = jnp.stack([padi(pos_r), padi(neg_r)], axis=-1)
    ridx = jnp.pad(ridx, ((0, 0), (0, L - 2)))
    ridx = ridx.reshape(NCHUNK_PAD, C, L)

    mesh = plsc.VectorSubcoreMesh(core_axis_name="c", subcore_axis_name="s",
                                  num_cores=NC, num_subcores=NS)
    parts = pl.kernel(
        _body,
        out_type=jax.ShapeDtypeStruct((NW, L), jnp.float32),
        mesh=mesh,
        compiler_params=pltpu.CompilerParams(needs_layout_passes=False),
        scratch_types=[
            pltpu.VMEM((2, 4, C), jnp.int32),
            pltpu.VMEM((2, C, L), jnp.int32),
            pltpu.VMEM((2, 8, C, DIM), jnp.float32),
            pltpu.VMEM((N_REL, DIM), jnp.float32),
            pltpu.VMEM((N_REL, DIM), jnp.float32),
            pltpu.VMEM((N_REL, L), jnp.float32),
            pltpu.VMEM((L,), jnp.float32),
            pltpu.SemaphoreType.DMA,
            pltpu.SemaphoreType.DMA,
            pltpu.SemaphoreType.DMA,
            pltpu.SemaphoreType.DMA,
        ],
    )(comb, ridx, entities_user_emb, entities_user_proj_emb,
      rel_emb, rel_proj_emb)

    s = jnp.sum(parts, axis=0)
    Ef = jnp.float32(E)
    return (s[0] + s[1]) / Ef + s[2] / (4.0 * Ef) + s[3] / (2.0 * Ef)
